# Initial kernel scaffold; baseline (speedup 1.0000x reference)
import jax
import jax.numpy as jnp
from jax.experimental import pallas as pl


def kernel(tokens, table):
    raise NotImplementedError("write your pallas kernel here")


# speedup vs baseline: 3.1850x; 3.1850x over previous
import functools
import math

import jax
import jax.numpy as jnp
from jax import lax
from jax.experimental import pallas as pl
from jax.experimental.pallas import tpu as pltpu
from jax.experimental.pallas import tpu_sc as plsc

EMB = 128
SCALE = math.sqrt(float(EMB))
NC = 2
NS = 16
NW = NC * NS
LANES = 16
CHUNK = 128


@functools.partial(jax.jit, static_argnums=(2,))
def _embed(tokens_flat, table, vocab):
    b = tokens_flat.shape[0]
    bpw = b // NW
    nchunks = bpw // CHUNK
    mesh = plsc.VectorSubcoreMesh(core_axis_name="c", subcore_axis_name="s")

    @functools.partial(
        pl.kernel,
        mesh=mesh,
        out_type=jax.ShapeDtypeStruct((b, EMB), jnp.float32),
        scratch_types=[
            pltpu.VMEM((CHUNK,), jnp.int32),
            pltpu.VMEM((CHUNK, EMB), jnp.float32),
            pltpu.SemaphoreType.DMA,
        ],
    )
    def k(tok_hbm, table_hbm, out_hbm, idx_v, rows_v, sem):
        wid = lax.axis_index("s") * NC + lax.axis_index("c")
        base = wid * bpw

        def chunk_body(g, carry):
            off = base + g * CHUNK
            pltpu.sync_copy(tok_hbm.at[pl.ds(off, CHUNK)], idx_v)
            for i in range(CHUNK // LANES):
                s = pl.ds(i * LANES, LANES)
                idx_v[s] = jnp.minimum(idx_v[s], vocab - 1)
            pltpu.async_copy(table_hbm.at[idx_v], rows_v, sem).wait()

            def scale_body(r, c2):
                for j in range(EMB // LANES):
                    s = pl.ds(j * LANES, LANES)
                    rows_v[r, s] = rows_v[r, s] * SCALE
                return c2

            lax.fori_loop(0, CHUNK, scale_body, 0, unroll=2)
            pltpu.sync_copy(rows_v, out_hbm.at[pl.ds(off, CHUNK)])
            return carry

        lax.fori_loop(0, nchunks, chunk_body, 0)

    return k(tokens_flat, table)


def kernel(tokens, table):
    b0, b1 = tokens.shape
    out = _embed(tokens.reshape(b0 * b1), table, table.shape[0])
    return out.reshape(b0, b1, EMB)
